# baseline (speedup 1.0000x reference)
import functools

import jax
import jax.numpy as jnp
from jax import lax
from jax.experimental import pallas as pl
from jax.experimental.pallas import tpu as pltpu
from jax.experimental.pallas import tpu_sc as plsc

_B, _S, _H = 4, 8192, 1024
_K, _D = 8192, 64
_N = _B * _S

_NC, _NS = 2, 16
_NW = _NC * _NS
_DP = 128

_TOK_BLOCK = 2048
_N_CHUNKS = 1
_CHUNK_TOKENS = _N // _N_CHUNKS
_BLOCKS_PER_CHUNK = _CHUNK_TOKENS // _TOK_BLOCK


def _sc_gather(table_padded, codes_chunk):
    n_rows = codes_chunk.shape[0]
    rows_per_w = n_rows // _NW
    sc_chunk = min(rows_per_w, 256)
    n_sc_chunks = rows_per_w // sc_chunk
    mesh = plsc.VectorSubcoreMesh(core_axis_name="c", subcore_axis_name="s")

    @functools.partial(
        pl.kernel,
        mesh=mesh,
        out_type=jax.ShapeDtypeStruct((n_rows, _DP), jnp.float32),
        scratch_types=[
            pltpu.VMEM((rows_per_w,), jnp.int32),
            pltpu.VMEM((sc_chunk, _DP), jnp.float32),
            pltpu.VMEM((sc_chunk, _DP), jnp.float32),
            pltpu.SemaphoreType.DMA,
            pltpu.SemaphoreType.DMA,
        ],
    )
    def gather_kernel(table_hbm, idx_hbm, out_hbm, idx_v, rows_a, rows_b,
                      sem_a, sem_b):
        wid = lax.axis_index("s") * _NC + lax.axis_index("c")
        base = wid * rows_per_w
        pltpu.sync_copy(idx_hbm.at[pl.ds(base, rows_per_w)], idx_v)

        bufs = (rows_a, rows_b)
        sems = (sem_a, sem_b)
        copies = [None] * n_sc_chunks
        copies[0] = pltpu.async_copy(
            table_hbm.at[idx_v.at[pl.ds(0, sc_chunk)]], bufs[0], sems[0])
        for j in range(n_sc_chunks):
            if j + 1 < n_sc_chunks:
                copies[j + 1] = pltpu.async_copy(
                    table_hbm.at[idx_v.at[pl.ds((j + 1) * sc_chunk,
                                                sc_chunk)]],
                    bufs[(j + 1) % 2], sems[(j + 1) % 2])
            copies[j].wait()
            pltpu.sync_copy(bufs[j % 2],
                            out_hbm.at[pl.ds(base + j * sc_chunk, sc_chunk)])

    return gather_kernel(table_padded, codes_chunk)


def _tc_body(uncond_ref, embs_ref, w_ref, b_ref, g_ref, out_ref):
    proj = jnp.dot(embs_ref[...].astype(jnp.bfloat16),
                   w_ref[...].astype(jnp.bfloat16),
                   preferred_element_type=jnp.float32)
    out_ref[...] = uncond_ref[...] + (proj + b_ref[...]) * g_ref[...]


def _tc_body_aliased(prev_ref, uncond_ref, embs_ref, w_ref, b_ref, g_ref,
                     out_ref):
    del prev_ref
    _tc_body(uncond_ref, embs_ref, w_ref, b_ref, g_ref, out_ref)


def _tc_fused_chunk(prev, uncond2d, embs_c, w_padded, b_proj2d, gate, chunk):
    blk0 = chunk * _BLOCKS_PER_CHUNK
    data_specs = [
        pl.BlockSpec((_TOK_BLOCK, _H), lambda i: (blk0 + i, 0)),
        pl.BlockSpec((_TOK_BLOCK, _DP), lambda i: (i, 0)),
        pl.BlockSpec((_DP, _H), lambda i: (0, 0)),
        pl.BlockSpec((1, _H), lambda i: (0, 0)),
        pl.BlockSpec((1, _H), lambda i: (0, 0)),
    ]
    common = dict(
        grid=(_BLOCKS_PER_CHUNK,),
        out_specs=pl.BlockSpec((_TOK_BLOCK, _H), lambda i: (blk0 + i, 0)),
        out_shape=jax.ShapeDtypeStruct((_N, _H), jnp.float32),
        compiler_params=pltpu.CompilerParams(
            dimension_semantics=("arbitrary",),
        ),
    )
    if prev is None:
        return pl.pallas_call(
            _tc_body, in_specs=data_specs, **common,
        )(uncond2d, embs_c, w_padded, b_proj2d, gate)
    return pl.pallas_call(
        _tc_body_aliased,
        in_specs=[pl.BlockSpec(memory_space=pltpu.MemorySpace.HBM)]
        + data_specs,
        input_output_aliases={0: 0},
        **common,
    )(prev, uncond2d, embs_c, w_padded, b_proj2d, gate)


def kernel(unconditioned, codes, codebook, W_proj, b_proj, gate):
    codes_flat = codes.reshape(_N)
    table_padded = jnp.pad(codebook, ((0, 0), (0, _DP - _D)))
    w_padded = jnp.pad(W_proj, ((0, _DP - _D), (0, 0)))
    b_proj2d = b_proj.reshape(1, _H)
    uncond2d = unconditioned.reshape(_N, _H)

    embs = [
        _sc_gather(
            table_padded,
            lax.slice(codes_flat, (c * _CHUNK_TOKENS,),
                      ((c + 1) * _CHUNK_TOKENS,)),
        )
        for c in range(_N_CHUNKS)
    ]
    out = None
    for c in range(_N_CHUNKS):
        out = _tc_fused_chunk(out, uncond2d, embs[c], w_padded, b_proj2d,
                              gate, c)
    return out.reshape(_B, _S, _H)


# speedup vs baseline: 1.0031x; 1.0031x over previous
import functools

import jax
import jax.numpy as jnp
from jax import lax
from jax.experimental import pallas as pl
from jax.experimental.pallas import tpu as pltpu
from jax.experimental.pallas import tpu_sc as plsc

_B, _S, _H = 4, 8192, 1024
_K, _D = 8192, 64
_N = _B * _S

_NC, _NS = 2, 16
_NW = _NC * _NS
_DP = 128
_ROWS_PER_W = _N // _NW
_SC_CHUNK = 512

_TOK_BLOCK = 2048


def _sc_gather(table_padded, codes_flat):
    mesh = plsc.VectorSubcoreMesh(core_axis_name="c", subcore_axis_name="s")

    @functools.partial(
        pl.kernel,
        mesh=mesh,
        out_type=jax.ShapeDtypeStruct((_N, _DP), jnp.float32),
        scratch_types=[
            pltpu.VMEM((_ROWS_PER_W,), jnp.int32),
            pltpu.VMEM((_SC_CHUNK, _DP), jnp.float32),
            pltpu.SemaphoreType.DMA,
        ],
    )
    def gather_kernel(table_hbm, idx_hbm, out_hbm, idx_v, rows_v, sem):
        wid = lax.axis_index("s") * _NC + lax.axis_index("c")
        base = wid * _ROWS_PER_W
        pltpu.sync_copy(idx_hbm.at[pl.ds(base, _ROWS_PER_W)], idx_v)

        @pl.loop(0, _ROWS_PER_W, step=_SC_CHUNK)
        def _(r):
            pltpu.async_copy(
                table_hbm.at[idx_v.at[pl.ds(r, _SC_CHUNK)]], rows_v, sem
            ).wait()
            pltpu.sync_copy(rows_v, out_hbm.at[pl.ds(base + r, _SC_CHUNK)])

    return gather_kernel(table_padded, codes_flat)


def _tc_body(uncond_ref, embs_ref, w_ref, b_ref, g_ref, out_ref):
    proj = jnp.dot(embs_ref[...].astype(jnp.bfloat16),
                   w_ref[...].astype(jnp.bfloat16),
                   preferred_element_type=jnp.float32)
    out_ref[...] = uncond_ref[...] + (proj + b_ref[...]) * g_ref[...]


def _tc_fused(uncond2d, embs, w_padded, b_proj2d, gate):
    return pl.pallas_call(
        _tc_body,
        grid=(_N // _TOK_BLOCK,),
        in_specs=[
            pl.BlockSpec((_TOK_BLOCK, _H), lambda i: (i, 0)),
            pl.BlockSpec((_TOK_BLOCK, _DP), lambda i: (i, 0)),
            pl.BlockSpec((_DP, _H), lambda i: (0, 0)),
            pl.BlockSpec((1, _H), lambda i: (0, 0)),
            pl.BlockSpec((1, _H), lambda i: (0, 0)),
        ],
        out_specs=pl.BlockSpec((_TOK_BLOCK, _H), lambda i: (i, 0)),
        out_shape=jax.ShapeDtypeStruct((_N, _H), jnp.float32),
        compiler_params=pltpu.CompilerParams(
            dimension_semantics=("arbitrary",),
        ),
    )(uncond2d, embs, w_padded, b_proj2d, gate)


def kernel(unconditioned, codes, codebook, W_proj, b_proj, gate):
    codes_flat = codes.reshape(_N)
    table_padded = jnp.pad(codebook, ((0, 0), (0, _DP - _D)))
    w_padded = jnp.pad(W_proj, ((0, _DP - _D), (0, 0)))
    embs = _sc_gather(table_padded, codes_flat)
    uncond2d = unconditioned.reshape(_N, _H)
    out = _tc_fused(uncond2d, embs, w_padded, b_proj.reshape(1, _H), gate)
    return out.reshape(_B, _S, _H)
